# baseline (speedup 1.0000x reference)
import jax
import jax.numpy as jnp
from jax.experimental import pallas as pl
from jax.experimental.pallas import tpu as pltpu

_N = 12
_SPLIT = 2


def _spec_kernel(t_ref, fl_ref, f_ref, e_ref, x_hbm, o_hbm,
                 eye, xbuf, obuf, in_sems, out_sems):
    C, R, D = x_hbm.shape

    def in_copy(i, s):
        return pltpu.make_async_copy(x_hbm.at[i], xbuf.at[s], in_sems.at[s])

    def out_copy(i, s):
        return pltpu.make_async_copy(obuf.at[s], o_hbm.at[i], out_sems.at[s])

    for s in range(_N):
        in_copy(s, s).start()

    rows = jax.lax.broadcasted_iota(jnp.int32, (R, R), 0)
    cols = jax.lax.broadcasted_iota(jnp.int32, (R, R), 1)
    eye[...] = jnp.where(rows == cols, jnp.float32(1), jnp.float32(0))

    e = e_ref[...]

    def step(i, carry):
        s = jax.lax.rem(i, _N)

        trow = jnp.where(jnp.logical_and(t_ref[pl.ds(i, 1)] != 0,
                                         fl_ref[pl.ds(i, 1)] != 0),
                         jnp.float32(1), jnp.float32(0))
        tcol = jax.lax.dot_general(
            eye[...], trow, (((1,), (1,)), ((), ())),
            preferred_element_type=jnp.float32)
        t = tcol != 0.0
        f = f_ref[pl.ds(jax.lax.div(i, _SPLIT), 1)] != 0

        in_copy(i, s).wait()

        @pl.when(i >= _N)
        def _():
            out_copy(i - _N, s).wait()

        obuf[s] = jnp.where(f, jnp.float32(0.0), jnp.where(t, e, xbuf[s]))
        out_copy(i, s).start()

        @pl.when(i + _N < C)
        def _():
            in_copy(i + _N, s).start()

        return carry

    jax.lax.fori_loop(0, C, step, 0)
    for s in range(_N):
        out_copy(C - _N + s, jax.lax.rem(C - _N + s, _N)).wait()


def kernel(x, masked_spec_embed, mask_time, flip_mask, mask_feature):
    B, L, D = x.shape
    C, R = B * _SPLIT, L // _SPLIT
    x2 = x.reshape(C, R, D)
    tm = mask_time.reshape(C, R)
    fl = flip_mask.reshape(C, R)
    e = masked_spec_embed.reshape(1, D).astype(x.dtype)

    out = pl.pallas_call(
        _spec_kernel,
        in_specs=[
            pl.BlockSpec((C, R), lambda: (0, 0)),
            pl.BlockSpec((C, R), lambda: (0, 0)),
            pl.BlockSpec((B, D), lambda: (0, 0)),
            pl.BlockSpec((1, D), lambda: (0, 0)),
            pl.BlockSpec(memory_space=pl.ANY),
        ],
        out_specs=pl.BlockSpec(memory_space=pl.ANY),
        out_shape=jax.ShapeDtypeStruct((C, R, D), x.dtype),
        scratch_shapes=[
            pltpu.VMEM((R, R), jnp.float32),
            pltpu.VMEM((_N, R, D), x.dtype),
            pltpu.VMEM((_N, R, D), x.dtype),
            pltpu.SemaphoreType.DMA((_N,)),
            pltpu.SemaphoreType.DMA((_N,)),
        ],
    )(tm, fl, mask_feature, e, x2)
    return out.reshape(B, L, D)


# speedup vs baseline: 1.0154x; 1.0154x over previous
import jax
import jax.numpy as jnp
from jax.experimental import pallas as pl
from jax.experimental.pallas import tpu as pltpu

_N = 6


def _spec_kernel(t_hbm, fl_hbm, f_hbm, e_hbm, x_hbm, o_hbm,
                 tvm, flvm, fvm, evm, eye, xbuf, obuf,
                 m_sem, in_sems, out_sems):
    B, L, D = x_hbm.shape
    H = L // 2

    def in_copy(i, s):
        return pltpu.make_async_copy(x_hbm.at[i], xbuf.at[s], in_sems.at[s])

    def out_half(i, s, h):
        sl = pl.ds(h * H, H)
        return pltpu.make_async_copy(obuf.at[s, sl], o_hbm.at[i, sl],
                                     out_sems.at[s])

    mask_copies = [
        pltpu.make_async_copy(t_hbm, tvm, m_sem),
        pltpu.make_async_copy(fl_hbm, flvm, m_sem),
        pltpu.make_async_copy(f_hbm, fvm, m_sem),
        pltpu.make_async_copy(e_hbm, evm, m_sem),
    ]
    for c in mask_copies:
        c.start()
    for s in range(_N):
        in_copy(s, s).start()

    rows = jax.lax.broadcasted_iota(jnp.int32, (L, L), 0)
    cols = jax.lax.broadcasted_iota(jnp.int32, (L, L), 1)
    eye[...] = jnp.where(rows == cols, jnp.float32(1), jnp.float32(0))

    for c in mask_copies:
        c.wait()
    e = evm[...]

    tf = (tvm[...].astype(jnp.float32) *
          flvm[...].astype(jnp.float32))
    tmat = jax.lax.dot_general(
        eye[...], tf, (((1,), (1,)), ((), ())),
        preferred_element_type=jnp.float32)
    fm = fvm[...].astype(jnp.float32)

    def step(i, carry):
        s = jax.lax.rem(i, _N)

        ohl = (jax.lax.broadcasted_iota(jnp.int32, (1, B), 1) == i
               ).astype(jnp.float32)
        t = jnp.sum(tmat * ohl, axis=1, keepdims=True) != 0.0
        ohs = (jax.lax.broadcasted_iota(jnp.int32, (B, 1), 0) == i
               ).astype(jnp.float32)
        f = jnp.sum(fm * ohs, axis=0, keepdims=True) != 0.0

        in_copy(i, s).wait()

        @pl.when(i >= _N)
        def _():
            out_half(i - _N, s, 0).wait()
            out_half(i - _N, s, 1).wait()

        obuf[s] = jnp.where(f, jnp.float32(0.0), jnp.where(t, e, xbuf[s]))
        out_half(i, s, 0).start()
        out_half(i, s, 1).start()

        @pl.when(i + _N < B)
        def _():
            in_copy(i + _N, s).start()

        return carry

    jax.lax.fori_loop(0, B, step, 0)
    for k in range(_N):
        i = B - _N + k
        s = jax.lax.rem(i, _N)
        out_half(i, s, 0).wait()
        out_half(i, s, 1).wait()


def kernel(x, masked_spec_embed, mask_time, flip_mask, mask_feature):
    B, L, D = x.shape
    e = masked_spec_embed.reshape(1, D).astype(x.dtype)

    f = pl.pallas_call(
        _spec_kernel,
        in_specs=[
            pl.BlockSpec(memory_space=pl.ANY),
            pl.BlockSpec(memory_space=pl.ANY),
            pl.BlockSpec(memory_space=pl.ANY),
            pl.BlockSpec(memory_space=pl.ANY),
            pl.BlockSpec(memory_space=pl.ANY),
        ],
        out_specs=pl.BlockSpec(memory_space=pl.ANY),
        out_shape=jax.ShapeDtypeStruct((B, L, D), x.dtype),
        scratch_shapes=[
            pltpu.VMEM((B, L), jnp.int8),
            pltpu.VMEM((B, L), jnp.int8),
            pltpu.VMEM((B, D), jnp.int8),
            pltpu.VMEM((1, D), x.dtype),
            pltpu.VMEM((L, L), jnp.float32),
            pltpu.VMEM((_N, L, D), x.dtype),
            pltpu.VMEM((_N, L, D), x.dtype),
            pltpu.SemaphoreType.DMA,
            pltpu.SemaphoreType.DMA((_N,)),
            pltpu.SemaphoreType.DMA((_N,)),
        ],
    )
    bc = lambda m: m.view(jnp.int8)
    return f(bc(mask_time), bc(flip_mask), bc(mask_feature), e, x)


# speedup vs baseline: 1.0156x; 1.0002x over previous
import jax
import jax.numpy as jnp
from jax.experimental import pallas as pl
from jax.experimental.pallas import tpu as pltpu

_N = 6


def _spec_kernel(t_hbm, fl_hbm, f_hbm, e_hbm, x_hbm, o_hbm,
                 tvm, flvm, fvm, evm, eye, xbuf, obuf,
                 m_sem, in_sems, out_sems):
    B, L, D = x_hbm.shape
    H = L // 2

    def in_copy(i, s):
        return pltpu.make_async_copy(x_hbm.at[i], xbuf.at[s], in_sems.at[s])

    def out_half(i, s, h):
        sl = pl.ds(h * H, H)
        return pltpu.make_async_copy(obuf.at[s, sl], o_hbm.at[i, sl],
                                     out_sems.at[s])

    mask_copies = [
        pltpu.make_async_copy(t_hbm, tvm, m_sem),
        pltpu.make_async_copy(fl_hbm, flvm, m_sem),
        pltpu.make_async_copy(f_hbm, fvm, m_sem),
        pltpu.make_async_copy(e_hbm, evm, m_sem),
    ]
    for c in mask_copies:
        c.start()
    for s in range(_N):
        in_copy(s, s).start()

    rows = jax.lax.broadcasted_iota(jnp.int32, (L, L), 0)
    cols = jax.lax.broadcasted_iota(jnp.int32, (L, L), 1)
    eye[...] = jnp.where(rows == cols, jnp.float32(1), jnp.float32(0))

    for c in mask_copies:
        c.wait()
    e = evm[...]

    tf = (tvm[...].astype(jnp.float32) *
          flvm[...].astype(jnp.float32))
    tmat = jax.lax.dot_general(
        eye[...], tf, (((1,), (1,)), ((), ())),
        preferred_element_type=jnp.float32)
    fm = fvm[...].astype(jnp.float32)

    def step(j, carry):
        for k in range(2):
            i = 2 * j + k
            s = jax.lax.rem(i, _N)

            ohl = (jax.lax.broadcasted_iota(jnp.int32, (1, B), 1) == i
                   ).astype(jnp.float32)
            t = jnp.sum(tmat * ohl, axis=1, keepdims=True) != 0.0
            ohs = (jax.lax.broadcasted_iota(jnp.int32, (B, 1), 0) == i
                   ).astype(jnp.float32)
            f = jnp.sum(fm * ohs, axis=0, keepdims=True) != 0.0

            in_copy(i, s).wait()

            @pl.when(i >= _N)
            def _():
                out_half(i - _N, s, 0).wait()
                out_half(i - _N, s, 1).wait()

            obuf[s] = jnp.where(f, jnp.float32(0.0), jnp.where(t, e, xbuf[s]))
            out_half(i, s, 0).start()
            out_half(i, s, 1).start()

            @pl.when(i + _N < B)
            def _():
                in_copy(i + _N, s).start()

        return carry

    jax.lax.fori_loop(0, B // 2, step, 0)
    for k in range(_N):
        i = B - _N + k
        s = jax.lax.rem(i, _N)
        out_half(i, s, 0).wait()
        out_half(i, s, 1).wait()


def kernel(x, masked_spec_embed, mask_time, flip_mask, mask_feature):
    B, L, D = x.shape
    e = masked_spec_embed.reshape(1, D).astype(x.dtype)

    f = pl.pallas_call(
        _spec_kernel,
        in_specs=[
            pl.BlockSpec(memory_space=pl.ANY),
            pl.BlockSpec(memory_space=pl.ANY),
            pl.BlockSpec(memory_space=pl.ANY),
            pl.BlockSpec(memory_space=pl.ANY),
            pl.BlockSpec(memory_space=pl.ANY),
        ],
        out_specs=pl.BlockSpec(memory_space=pl.ANY),
        out_shape=jax.ShapeDtypeStruct((B, L, D), x.dtype),
        scratch_shapes=[
            pltpu.VMEM((B, L), jnp.int8),
            pltpu.VMEM((B, L), jnp.int8),
            pltpu.VMEM((B, D), jnp.int8),
            pltpu.VMEM((1, D), x.dtype),
            pltpu.VMEM((L, L), jnp.float32),
            pltpu.VMEM((_N, L, D), x.dtype),
            pltpu.VMEM((_N, L, D), x.dtype),
            pltpu.SemaphoreType.DMA,
            pltpu.SemaphoreType.DMA((_N,)),
            pltpu.SemaphoreType.DMA((_N,)),
        ],
    )
    bc = lambda m: m.view(jnp.int8)
    return f(bc(mask_time), bc(flip_mask), bc(mask_feature), e, x)


# speedup vs baseline: 1.0173x; 1.0016x over previous
import jax
import jax.numpy as jnp
from jax.experimental import pallas as pl
from jax.experimental.pallas import tpu as pltpu

_N = 6


def _spec_kernel(t_hbm, fl_hbm, f_hbm, e_hbm, x_hbm, o_hbm,
                 tvm, flvm, fvm, evm, eye, xbuf, obuf,
                 m_sem, h1_sem, in_sems, out_sems):
    B, L, D = x_hbm.shape
    H = L // 2

    def in_copy(i, s):
        return pltpu.make_async_copy(x_hbm.at[i], xbuf.at[s], in_sems.at[s])

    def out_half(i, s, h):
        sl = pl.ds(h * H, H)
        return pltpu.make_async_copy(obuf.at[s, sl], o_hbm.at[i, sl],
                                     out_sems.at[s])

    mask_copies = [
        pltpu.make_async_copy(t_hbm, tvm, m_sem),
        pltpu.make_async_copy(fl_hbm, flvm, m_sem),
        pltpu.make_async_copy(f_hbm, fvm, m_sem),
        pltpu.make_async_copy(e_hbm, evm, m_sem),
    ]
    for c in mask_copies:
        c.start()
    def in_half(s, h, sem):
        sl = pl.ds(h * H, H)
        return pltpu.make_async_copy(x_hbm.at[s, sl], xbuf.at[s, sl], sem)

    c0h0 = in_half(0, 0, in_sems.at[0])
    c0h1 = in_half(0, 1, h1_sem)
    c0h0.start()
    c0h1.start()
    for s in range(1, _N):
        in_copy(s, s).start()

    rows = jax.lax.broadcasted_iota(jnp.int32, (L, L), 0)
    cols = jax.lax.broadcasted_iota(jnp.int32, (L, L), 1)
    eye[...] = jnp.where(rows == cols, jnp.float32(1), jnp.float32(0))

    for c in mask_copies:
        c.wait()
    e = evm[...]

    tf = (tvm[...].astype(jnp.float32) *
          flvm[...].astype(jnp.float32))
    tmat = jax.lax.dot_general(
        eye[...], tf, (((1,), (1,)), ((), ())),
        preferred_element_type=jnp.float32)
    fm = fvm[...].astype(jnp.float32)

    def sample_masks(i):
        ohl = (jax.lax.broadcasted_iota(jnp.int32, (1, B), 1) == i
               ).astype(jnp.float32)
        t = jnp.sum(tmat * ohl, axis=1, keepdims=True) != 0.0
        ohs = (jax.lax.broadcasted_iota(jnp.int32, (B, 1), 0) == i
               ).astype(jnp.float32)
        f = jnp.sum(fm * ohs, axis=0, keepdims=True) != 0.0
        return t, f

    t0, f0 = sample_masks(0)
    c0h0.wait()
    obuf[0, : H] = jnp.where(f0, jnp.float32(0.0),
                             jnp.where(t0[:H], e, xbuf[0, : H]))
    out_half(0, 0, 0).start()
    c0h1.wait()
    obuf[0, H:] = jnp.where(f0, jnp.float32(0.0),
                            jnp.where(t0[H:], e, xbuf[0, H:]))
    out_half(0, 0, 1).start()
    in_copy(_N, 0).start()

    def step(j, carry):
        for k in range(2):
            i = 2 * j + k + 1
            s = jax.lax.rem(i, _N)

            t, f = sample_masks(i)

            in_copy(i, s).wait()

            @pl.when(i >= _N)
            def _():
                out_half(i - _N, s, 0).wait()
                out_half(i - _N, s, 1).wait()

            obuf[s] = jnp.where(f, jnp.float32(0.0), jnp.where(t, e, xbuf[s]))
            out_half(i, s, 0).start()
            out_half(i, s, 1).start()

            @pl.when(i + _N < B)
            def _():
                in_copy(i + _N, s).start()

        return carry

    jax.lax.fori_loop(0, (B - 1) // 2, step, 0)
    i = B - 1
    s = jax.lax.rem(i, _N)
    t, f = sample_masks(i)
    in_copy(i, s).wait()
    out_half(i - _N, s, 0).wait()
    out_half(i - _N, s, 1).wait()
    obuf[s] = jnp.where(f, jnp.float32(0.0), jnp.where(t, e, xbuf[s]))
    out_half(i, s, 0).start()
    out_half(i, s, 1).start()
    for k in range(_N):
        i = B - _N + k
        s = jax.lax.rem(i, _N)
        out_half(i, s, 0).wait()
        out_half(i, s, 1).wait()


def kernel(x, masked_spec_embed, mask_time, flip_mask, mask_feature):
    B, L, D = x.shape
    e = masked_spec_embed.reshape(1, D).astype(x.dtype)

    f = pl.pallas_call(
        _spec_kernel,
        in_specs=[
            pl.BlockSpec(memory_space=pl.ANY),
            pl.BlockSpec(memory_space=pl.ANY),
            pl.BlockSpec(memory_space=pl.ANY),
            pl.BlockSpec(memory_space=pl.ANY),
            pl.BlockSpec(memory_space=pl.ANY),
        ],
        out_specs=pl.BlockSpec(memory_space=pl.ANY),
        out_shape=jax.ShapeDtypeStruct((B, L, D), x.dtype),
        scratch_shapes=[
            pltpu.VMEM((B, L), jnp.int8),
            pltpu.VMEM((B, L), jnp.int8),
            pltpu.VMEM((B, D), jnp.int8),
            pltpu.VMEM((1, D), x.dtype),
            pltpu.VMEM((L, L), jnp.float32),
            pltpu.VMEM((_N, L, D), x.dtype),
            pltpu.VMEM((_N, L, D), x.dtype),
            pltpu.SemaphoreType.DMA,
            pltpu.SemaphoreType.DMA,
            pltpu.SemaphoreType.DMA((_N,)),
            pltpu.SemaphoreType.DMA((_N,)),
        ],
    )
    bc = lambda m: m.view(jnp.int8)
    return f(bc(mask_time), bc(flip_mask), bc(mask_feature), e, x)


# speedup vs baseline: 1.0201x; 1.0027x over previous
import jax
import jax.numpy as jnp
from jax.experimental import pallas as pl
from jax.experimental.pallas import tpu as pltpu

_N = 9
_NO = 3


def _spec_kernel(t_hbm, fl_hbm, f_hbm, e_hbm, x_hbm, o_hbm,
                 tvm, flvm, fvm, evm, eye, xbuf, obuf,
                 m_sem, h1_sem, in_sems, out_sems):
    B, L, D = x_hbm.shape
    H = L // 2

    def in_copy(i, s):
        return pltpu.make_async_copy(x_hbm.at[i], xbuf.at[s], in_sems.at[s])

    def out_half(i, so, h):
        sl = pl.ds(h * H, H)
        return pltpu.make_async_copy(obuf.at[so, sl], o_hbm.at[i, sl],
                                     out_sems.at[so])

    mask_copies = [
        pltpu.make_async_copy(t_hbm, tvm, m_sem),
        pltpu.make_async_copy(fl_hbm, flvm, m_sem),
        pltpu.make_async_copy(f_hbm, fvm, m_sem),
        pltpu.make_async_copy(e_hbm, evm, m_sem),
    ]
    for c in mask_copies:
        c.start()
    def in_half(s, h, sem):
        sl = pl.ds(h * H, H)
        return pltpu.make_async_copy(x_hbm.at[s, sl], xbuf.at[s, sl], sem)

    c0h0 = in_half(0, 0, in_sems.at[0])
    c0h1 = in_half(0, 1, h1_sem)
    c0h0.start()
    c0h1.start()
    for s in range(1, _N):
        in_copy(s, s).start()

    rows = jax.lax.broadcasted_iota(jnp.int32, (L, L), 0)
    cols = jax.lax.broadcasted_iota(jnp.int32, (L, L), 1)
    eye[...] = jnp.where(rows == cols, jnp.float32(1), jnp.float32(0))

    for c in mask_copies:
        c.wait()
    e = evm[...]

    tf = (tvm[...].astype(jnp.float32) *
          flvm[...].astype(jnp.float32))
    tmat = jax.lax.dot_general(
        eye[...], tf, (((1,), (1,)), ((), ())),
        preferred_element_type=jnp.float32)
    fm = fvm[...].astype(jnp.float32)

    def sample_masks(i):
        ohl = (jax.lax.broadcasted_iota(jnp.int32, (1, B), 1) == i
               ).astype(jnp.float32)
        t = jnp.sum(tmat * ohl, axis=1, keepdims=True) != 0.0
        ohs = (jax.lax.broadcasted_iota(jnp.int32, (B, 1), 0) == i
               ).astype(jnp.float32)
        f = jnp.sum(fm * ohs, axis=0, keepdims=True) != 0.0
        return t, f

    t0, f0 = sample_masks(0)
    c0h0.wait()
    obuf[0, : H] = jnp.where(f0, jnp.float32(0.0),
                             jnp.where(t0[:H], e, xbuf[0, : H]))
    out_half(0, 0, 0).start()
    c0h1.wait()
    obuf[0, H:] = jnp.where(f0, jnp.float32(0.0),
                            jnp.where(t0[H:], e, xbuf[0, H:]))
    out_half(0, 0, 1).start()
    in_copy(_N, 0).start()

    def step(j, carry):
        for k in range(2):
            i = 2 * j + k + 1
            s = jax.lax.rem(i, _N)
            so = jax.lax.rem(i, _NO)

            t, f = sample_masks(i)

            in_copy(i, s).wait()

            @pl.when(i >= _NO)
            def _():
                out_half(i - _NO, so, 0).wait()
                out_half(i - _NO, so, 1).wait()

            obuf[so] = jnp.where(f, jnp.float32(0.0), jnp.where(t, e, xbuf[s]))
            out_half(i, so, 0).start()
            out_half(i, so, 1).start()

            @pl.when(i + _N < B)
            def _():
                in_copy(i + _N, s).start()

        return carry

    jax.lax.fori_loop(0, (B - 1) // 2, step, 0)
    i = B - 1
    s = jax.lax.rem(i, _N)
    so = jax.lax.rem(i, _NO)
    t, f = sample_masks(i)
    in_copy(i, s).wait()
    out_half(i - _NO, so, 0).wait()
    out_half(i - _NO, so, 1).wait()
    obuf[so] = jnp.where(f, jnp.float32(0.0), jnp.where(t, e, xbuf[s]))
    out_half(i, so, 0).start()
    out_half(i, so, 1).start()
    for k in range(_NO):
        i = B - _NO + k
        so = jax.lax.rem(i, _NO)
        out_half(i, so, 0).wait()
        out_half(i, so, 1).wait()


def kernel(x, masked_spec_embed, mask_time, flip_mask, mask_feature):
    B, L, D = x.shape
    e = masked_spec_embed.reshape(1, D).astype(x.dtype)

    f = pl.pallas_call(
        _spec_kernel,
        in_specs=[
            pl.BlockSpec(memory_space=pl.ANY),
            pl.BlockSpec(memory_space=pl.ANY),
            pl.BlockSpec(memory_space=pl.ANY),
            pl.BlockSpec(memory_space=pl.ANY),
            pl.BlockSpec(memory_space=pl.ANY),
        ],
        out_specs=pl.BlockSpec(memory_space=pl.ANY),
        out_shape=jax.ShapeDtypeStruct((B, L, D), x.dtype),
        scratch_shapes=[
            pltpu.VMEM((B, L), jnp.int8),
            pltpu.VMEM((B, L), jnp.int8),
            pltpu.VMEM((B, D), jnp.int8),
            pltpu.VMEM((1, D), x.dtype),
            pltpu.VMEM((L, L), jnp.float32),
            pltpu.VMEM((_N, L, D), x.dtype),
            pltpu.VMEM((_NO, L, D), x.dtype),
            pltpu.SemaphoreType.DMA,
            pltpu.SemaphoreType.DMA,
            pltpu.SemaphoreType.DMA((_N,)),
            pltpu.SemaphoreType.DMA((_NO,)),
        ],
    )
    bc = lambda m: m.view(jnp.int8)
    return f(bc(mask_time), bc(flip_mask), bc(mask_feature), e, x)


# speedup vs baseline: 1.0217x; 1.0016x over previous
import jax
import jax.numpy as jnp
from jax.experimental import pallas as pl
from jax.experimental.pallas import tpu as pltpu

_N = 10
_NO = 3


def _spec_kernel(t_hbm, fl_hbm, f_hbm, e_hbm, x_hbm, o_hbm,
                 tvm, flvm, fvm, evm, eye, xbuf, obuf,
                 m_sem, h1_sem, in_sems, out_sems):
    B, L, D = x_hbm.shape
    H = L // 2

    def in_copy(i, s):
        return pltpu.make_async_copy(x_hbm.at[i], xbuf.at[s], in_sems.at[s])

    def out_half(i, so, h):
        sl = pl.ds(h * H, H)
        return pltpu.make_async_copy(obuf.at[so, sl], o_hbm.at[i, sl],
                                     out_sems.at[so])

    mask_copies = [
        pltpu.make_async_copy(t_hbm, tvm, m_sem),
        pltpu.make_async_copy(fl_hbm, flvm, m_sem),
        pltpu.make_async_copy(f_hbm, fvm, m_sem),
        pltpu.make_async_copy(e_hbm, evm, m_sem),
    ]
    for c in mask_copies:
        c.start()
    def in_half(s, h, sem):
        sl = pl.ds(h * H, H)
        return pltpu.make_async_copy(x_hbm.at[s, sl], xbuf.at[s, sl], sem)

    c0h0 = in_half(0, 0, in_sems.at[0])
    c0h1 = in_half(0, 1, h1_sem)
    c0h0.start()
    c0h1.start()
    for s in range(1, _N):
        in_copy(s, s).start()

    rows = jax.lax.broadcasted_iota(jnp.int32, (L, L), 0)
    cols = jax.lax.broadcasted_iota(jnp.int32, (L, L), 1)
    eye[...] = jnp.where(rows == cols, jnp.float32(1), jnp.float32(0))

    for c in mask_copies:
        c.wait()
    e = evm[...]

    tf = (tvm[...].astype(jnp.float32) *
          flvm[...].astype(jnp.float32))
    tmat = jax.lax.dot_general(
        eye[...], tf, (((1,), (1,)), ((), ())),
        preferred_element_type=jnp.float32)
    fm = fvm[...].astype(jnp.float32)

    def sample_masks(i):
        ohl = (jax.lax.broadcasted_iota(jnp.int32, (1, B), 1) == i
               ).astype(jnp.float32)
        t = jnp.sum(tmat * ohl, axis=1, keepdims=True) != 0.0
        ohs = (jax.lax.broadcasted_iota(jnp.int32, (B, 1), 0) == i
               ).astype(jnp.float32)
        f = jnp.sum(fm * ohs, axis=0, keepdims=True) != 0.0
        return t, f

    t0, f0 = sample_masks(0)
    c0h0.wait()
    obuf[0, : H] = jnp.where(f0, jnp.float32(0.0),
                             jnp.where(t0[:H], e, xbuf[0, : H]))
    out_half(0, 0, 0).start()
    c0h1.wait()
    obuf[0, H:] = jnp.where(f0, jnp.float32(0.0),
                            jnp.where(t0[H:], e, xbuf[0, H:]))
    out_half(0, 0, 1).start()
    in_copy(_N, 0).start()

    def step(j, carry):
        for k in range(2):
            i = 2 * j + k + 1
            s = jax.lax.rem(i, _N)
            so = jax.lax.rem(i, _NO)

            t, f = sample_masks(i)

            in_copy(i, s).wait()

            @pl.when(i >= _NO)
            def _():
                out_half(i - _NO, so, 0).wait()
                out_half(i - _NO, so, 1).wait()

            obuf[so] = jnp.where(f, jnp.float32(0.0), jnp.where(t, e, xbuf[s]))
            out_half(i, so, 0).start()
            out_half(i, so, 1).start()

            @pl.when(i + _N < B)
            def _():
                in_copy(i + _N, s).start()

        return carry

    jax.lax.fori_loop(0, (B - 1) // 2, step, 0)
    i = B - 1
    s = jax.lax.rem(i, _N)
    so = jax.lax.rem(i, _NO)
    t, f = sample_masks(i)
    in_copy(i, s).wait()
    out_half(i - _NO, so, 0).wait()
    out_half(i - _NO, so, 1).wait()
    obuf[so] = jnp.where(f, jnp.float32(0.0), jnp.where(t, e, xbuf[s]))
    out_half(i, so, 0).start()
    out_half(i, so, 1).start()
    for k in range(_NO):
        i = B - _NO + k
        so = jax.lax.rem(i, _NO)
        out_half(i, so, 0).wait()
        out_half(i, so, 1).wait()


def kernel(x, masked_spec_embed, mask_time, flip_mask, mask_feature):
    B, L, D = x.shape
    e = masked_spec_embed.reshape(1, D).astype(x.dtype)

    f = pl.pallas_call(
        _spec_kernel,
        in_specs=[
            pl.BlockSpec(memory_space=pl.ANY),
            pl.BlockSpec(memory_space=pl.ANY),
            pl.BlockSpec(memory_space=pl.ANY),
            pl.BlockSpec(memory_space=pl.ANY),
            pl.BlockSpec(memory_space=pl.ANY),
        ],
        out_specs=pl.BlockSpec(memory_space=pl.ANY),
        out_shape=jax.ShapeDtypeStruct((B, L, D), x.dtype),
        scratch_shapes=[
            pltpu.VMEM((B, L), jnp.int8),
            pltpu.VMEM((B, L), jnp.int8),
            pltpu.VMEM((B, D), jnp.int8),
            pltpu.VMEM((1, D), x.dtype),
            pltpu.VMEM((L, L), jnp.float32),
            pltpu.VMEM((_N, L, D), x.dtype),
            pltpu.VMEM((_NO, L, D), x.dtype),
            pltpu.SemaphoreType.DMA,
            pltpu.SemaphoreType.DMA,
            pltpu.SemaphoreType.DMA((_N,)),
            pltpu.SemaphoreType.DMA((_NO,)),
        ],
    )
    bc = lambda m: m.view(jnp.int8)
    return f(bc(mask_time), bc(flip_mask), bc(mask_feature), e, x)
